# baseline (speedup 1.0000x reference)
import functools

import jax
import jax.numpy as jnp
from jax import lax
from jax.experimental import pallas as pl
from jax.experimental.pallas import tpu as pltpu
from jax.experimental.pallas import tpu_sc as plsc

NUM_SAMPLES_ = 32

_NC = 2
_NS = 16
_L = 16

_B, _CX, _N = 16, 3, 4096
_C = 256
_S = NUM_SAMPLES_
_G = _N // _S
_GS = _G * _S

_TILES = _NC * _NS
_CPT = _C // (_TILES // _B)
_RG = 4
_NGROUPS = _CPT // _RG


def _sc_body(points_hbm, xyz_hbm, idx2_hbm, idx1_hbm, idxrep_hbm,
             gpts_hbm, nxyz_hbm, gxyz_hbm,
             idx_v, in0, in1, out0, out1, xyz_v, gxyz_v, idx1_v, idxrep_v,
             nxyz_v, sin0, sin1, sout0, sout1):
    cid = lax.axis_index("c")
    sid = lax.axis_index("s")
    wid = sid * _NC + cid
    b = wid // 2
    c0 = (wid % 2) * _CPT

    pltpu.sync_copy(idx2_hbm.at[b], idx_v)

    roff = [jnp.full((_L,), r * _N, dtype=jnp.int32) for r in range(_RG)]
    ins, outs = [in0, in1], [out0, out1]
    sins, souts = [sin0, sin1], [sout0, sout1]

    def start_in(gg, bb):
        for r in range(_RG):
            pltpu.async_copy(points_hbm.at[b, c0 + gg * _RG + r],
                             ins[bb].at[pl.ds(r * _N, _N)], sins[bb])

    def wait_in(gg, bb):
        for r in range(_RG):
            pltpu.make_async_copy(points_hbm.at[b, c0 + gg * _RG + r],
                                  ins[bb].at[pl.ds(r * _N, _N)],
                                  sins[bb]).wait()

    def start_out(gg, bb):
        for r in range(_RG):
            pltpu.async_copy(outs[bb].at[pl.ds(r * _G, _G), :],
                             gpts_hbm.at[b, c0 + gg * _RG + r], souts[bb])

    def wait_out(gg, bb):
        for r in range(_RG):
            pltpu.make_async_copy(outs[bb].at[pl.ds(r * _G, _G), :],
                                  gpts_hbm.at[b, c0 + gg * _RG + r],
                                  souts[bb]).wait()

    start_in(0, 0)
    start_in(1, 1)

    def pipe(i, carry):
        for bb in range(2):
            gg = 2 * i + bb
            wait_in(gg, bb)

            @pl.when(gg >= 2)
            def _():
                wait_out(gg - 2, bb)

            def inner(j, c2, bb=bb):
                idxv = idx_v[pl.ds(j * _L, _L)]
                row = j // 2
                s0 = (j % 2) * _L
                for r in range(_RG):
                    outs[bb][r * _G + row, pl.ds(s0, _L)] = plsc.load_gather(
                        ins[bb], [idxv + roff[r]])
                return c2

            lax.fori_loop(0, _GS // _L, inner, 0, unroll=4)
            start_out(gg, bb)

            @pl.when(gg + 2 < _NGROUPS)
            def _():
                start_in(gg + 2, bb)
        return carry

    lax.fori_loop(0, _NGROUPS // 2, pipe, 0)
    wait_out(_NGROUPS - 2, 0)
    wait_out(_NGROUPS - 1, 1)

    @pl.when(wid < _B)
    def _():
        b2 = wid
        for k in range(_CX):
            pltpu.sync_copy(xyz_hbm.at[b2, k], xyz_v.at[pl.ds(k * _N, _N)])
        pltpu.sync_copy(idx1_hbm.at[b2], idx1_v)
        pltpu.sync_copy(idxrep_hbm.at[b2], idxrep_v)
        koff = [jnp.full((_L,), k * _N, dtype=jnp.int32) for k in range(_CX)]
        for j in range(_G // _L):
            idxv = idx1_v[pl.ds(j * _L, _L)]
            for k in range(_CX):
                nxyz_v[pl.ds(k * _G + j * _L, _L)] = plsc.load_gather(
                    xyz_v, [idxv + koff[k]])
        for k in range(_CX):
            pltpu.sync_copy(nxyz_v.at[pl.ds(k * _G, _G)], nxyz_hbm.at[b2, k])

        def gx_inner(j, c2):
            idxv = idxrep_v[pl.ds(j * _L, _L)]
            row = j // 2
            s0 = (j % 2) * _L
            for k in range(_CX):
                gxyz_v[k * _G + row, pl.ds(s0, _L)] = plsc.load_gather(
                    xyz_v, [idxv + koff[k]])
            return c2

        lax.fori_loop(0, _GS // _L, gx_inner, 0, unroll=4)
        for k in range(_CX):
            pltpu.sync_copy(gxyz_v.at[pl.ds(k * _G, _G), :],
                            gxyz_hbm.at[b2, k])


@jax.jit
def kernel(xyz, points):
    B, C, N = points.shape
    idx_key = jax.random.key(42)
    k1, k2 = jax.random.split(idx_key)
    idx1 = jax.random.randint(k1, (B, _G), 0, N).astype(jnp.int32)
    idx2 = jax.random.randint(k2, (B, _G, _S), 0, N).astype(jnp.int32)
    idx2f = idx2.reshape(B, _GS)
    idxrep = jnp.repeat(idx1, _S, axis=1)

    mesh = plsc.VectorSubcoreMesh(
        core_axis_name="c", subcore_axis_name="s",
        num_cores=_NC, num_subcores=_NS)
    run = pl.kernel(
        _sc_body,
        out_type=(
            jax.ShapeDtypeStruct((B, C, _G, _S), jnp.float32),
            jax.ShapeDtypeStruct((B, _CX, _G), jnp.float32),
            jax.ShapeDtypeStruct((B, _CX, _G, _S), jnp.float32),
        ),
        mesh=mesh,
        compiler_params=pltpu.CompilerParams(
            needs_layout_passes=False, use_tc_tiling_on_sc=False),
        scratch_types=[
            pltpu.VMEM((_GS,), jnp.int32),
            pltpu.VMEM((_RG * _N,), jnp.float32),
            pltpu.VMEM((_RG * _N,), jnp.float32),
            pltpu.VMEM((_RG * _G, _S), jnp.float32),
            pltpu.VMEM((_RG * _G, _S), jnp.float32),
            pltpu.VMEM((_CX * _N,), jnp.float32),
            pltpu.VMEM((_CX * _G, _S), jnp.float32),
            pltpu.VMEM((_G,), jnp.int32),
            pltpu.VMEM((_GS,), jnp.int32),
            pltpu.VMEM((_CX * _G,), jnp.float32),
            pltpu.SemaphoreType.DMA,
            pltpu.SemaphoreType.DMA,
            pltpu.SemaphoreType.DMA,
            pltpu.SemaphoreType.DMA,
        ],
    )
    grouped_points, new_xyz, grouped_xyz = run(points, xyz, idx2f, idx1, idxrep)
    return (new_xyz, grouped_xyz, grouped_points)


# speedup vs baseline: 1.8742x; 1.8742x over previous
import functools

import jax
import jax.numpy as jnp
from jax import lax
from jax.experimental import pallas as pl
from jax.experimental.pallas import tpu as pltpu
from jax.experimental.pallas import tpu_sc as plsc

NUM_SAMPLES_ = 32

_NC = 2
_NS = 16
_L = 16

_B, _CX, _N = 16, 3, 4096
_C = 256
_S = NUM_SAMPLES_
_G = _N // _S
_GS = _G * _S

_TILES = _NC * _NS
_CPT = _C // (_TILES // _B)
_RG = 4
_NGROUPS = _CPT // _RG


def _sc_body(points_hbm, xyz_hbm, idx2_hbm, idx1_hbm, idxrep_hbm,
             gpts_hbm, nxyz_hbm, gxyz_hbm,
             idx_v, in0, in1, out0, out1, xyz_v, gxyz_v, idx1_v, idxrep_v,
             nxyz_v, sin0, sin1, sout0, sout1):
    cid = lax.axis_index("c")
    sid = lax.axis_index("s")
    wid = sid * _NC + cid
    b = wid // 2
    c0 = (wid % 2) * _CPT

    pltpu.sync_copy(idx2_hbm.at[b], idx_v)

    roff = [jnp.full((_L,), r * _N, dtype=jnp.int32) for r in range(_RG)]
    ins, outs = [in0, in1], [out0, out1]
    sins, souts = [sin0, sin1], [sout0, sout1]

    def start_in(gg, bb):
        for r in range(_RG):
            pltpu.async_copy(points_hbm.at[b, c0 + gg * _RG + r],
                             ins[bb].at[pl.ds(r * _N, _N)], sins[bb])

    def wait_in(gg, bb):
        for r in range(_RG):
            pltpu.make_async_copy(points_hbm.at[b, c0 + gg * _RG + r],
                                  ins[bb].at[pl.ds(r * _N, _N)],
                                  sins[bb]).wait()

    def start_out(gg, bb):
        for r in range(_RG):
            pltpu.async_copy(outs[bb].at[pl.ds(r * _S, _S), :],
                             gpts_hbm.at[b, c0 + gg * _RG + r], souts[bb])

    def wait_out(gg, bb):
        for r in range(_RG):
            pltpu.make_async_copy(outs[bb].at[pl.ds(r * _S, _S), :],
                                  gpts_hbm.at[b, c0 + gg * _RG + r],
                                  souts[bb]).wait()

    start_in(0, 0)
    start_in(1, 1)

    def pipe(i, carry):
        for bb in range(2):
            gg = 2 * i + bb
            wait_in(gg, bb)

            @pl.when(gg >= 2)
            def _():
                wait_out(gg - 2, bb)

            def inner(j, c2, bb=bb):
                idxv = idx_v[pl.ds(j * _L, _L)]
                row = j // (_G // _L)
                g0 = (j % (_G // _L)) * _L
                for r in range(_RG):
                    outs[bb][r * _S + row, pl.ds(g0, _L)] = plsc.load_gather(
                        ins[bb], [idxv + roff[r]])
                return c2

            lax.fori_loop(0, _GS // _L, inner, 0, unroll=4)
            start_out(gg, bb)

            @pl.when(gg + 2 < _NGROUPS)
            def _():
                start_in(gg + 2, bb)
        return carry

    lax.fori_loop(0, _NGROUPS // 2, pipe, 0)
    wait_out(_NGROUPS - 2, 0)
    wait_out(_NGROUPS - 1, 1)

    @pl.when(wid < _B)
    def _():
        b2 = wid
        for k in range(_CX):
            pltpu.sync_copy(xyz_hbm.at[b2, k], xyz_v.at[pl.ds(k * _N, _N)])
        pltpu.sync_copy(idx1_hbm.at[b2], idx1_v)
        pltpu.sync_copy(idxrep_hbm.at[b2], idxrep_v)
        koff = [jnp.full((_L,), k * _N, dtype=jnp.int32) for k in range(_CX)]
        for j in range(_G // _L):
            idxv = idx1_v[pl.ds(j * _L, _L)]
            for k in range(_CX):
                nxyz_v[pl.ds(k * _G + j * _L, _L)] = plsc.load_gather(
                    xyz_v, [idxv + koff[k]])
        for k in range(_CX):
            pltpu.sync_copy(nxyz_v.at[pl.ds(k * _G, _G)], nxyz_hbm.at[b2, k])

        def gx_inner(j, c2):
            idxv = idxrep_v[pl.ds(j * _L, _L)]
            row = j // (_G // _L)
            g0 = (j % (_G // _L)) * _L
            for k in range(_CX):
                gxyz_v[k * _S + row, pl.ds(g0, _L)] = plsc.load_gather(
                    xyz_v, [idxv + koff[k]])
            return c2

        lax.fori_loop(0, _GS // _L, gx_inner, 0, unroll=4)
        for k in range(_CX):
            pltpu.sync_copy(gxyz_v.at[pl.ds(k * _S, _S), :],
                            gxyz_hbm.at[b2, k])


@jax.jit
def kernel(xyz, points):
    B, C, N = points.shape
    idx_key = jax.random.key(42)
    k1, k2 = jax.random.split(idx_key)
    idx1 = jax.random.randint(k1, (B, _G), 0, N).astype(jnp.int32)
    idx2 = jax.random.randint(k2, (B, _G, _S), 0, N).astype(jnp.int32)
    idx2f = idx2.transpose(0, 2, 1).reshape(B, _GS)
    idxrep = jnp.tile(idx1, (1, _S))

    mesh = plsc.VectorSubcoreMesh(
        core_axis_name="c", subcore_axis_name="s",
        num_cores=_NC, num_subcores=_NS)
    run = pl.kernel(
        _sc_body,
        out_type=(
            jax.ShapeDtypeStruct((B, C, _S, _G), jnp.float32),
            jax.ShapeDtypeStruct((B, _CX, _G), jnp.float32),
            jax.ShapeDtypeStruct((B, _CX, _S, _G), jnp.float32),
        ),
        mesh=mesh,
        compiler_params=pltpu.CompilerParams(
            needs_layout_passes=False, use_tc_tiling_on_sc=False),
        scratch_types=[
            pltpu.VMEM((_GS,), jnp.int32),
            pltpu.VMEM((_RG * _N,), jnp.float32),
            pltpu.VMEM((_RG * _N,), jnp.float32),
            pltpu.VMEM((_RG * _S, _G), jnp.float32),
            pltpu.VMEM((_RG * _S, _G), jnp.float32),
            pltpu.VMEM((_CX * _N,), jnp.float32),
            pltpu.VMEM((_CX * _S, _G), jnp.float32),
            pltpu.VMEM((_G,), jnp.int32),
            pltpu.VMEM((_GS,), jnp.int32),
            pltpu.VMEM((_CX * _G,), jnp.float32),
            pltpu.SemaphoreType.DMA,
            pltpu.SemaphoreType.DMA,
            pltpu.SemaphoreType.DMA,
            pltpu.SemaphoreType.DMA,
        ],
    )
    gpts_t, new_xyz, gxyz_t = run(points, xyz, idx2f, idx1, idxrep)
    grouped_points = gpts_t.transpose(0, 1, 3, 2)
    grouped_xyz = gxyz_t.transpose(0, 1, 3, 2)
    return (new_xyz, grouped_xyz, grouped_points)


# speedup vs baseline: 2.0396x; 1.0883x over previous
import functools

import jax
import jax.numpy as jnp
from jax import lax
from jax.experimental import pallas as pl
from jax.experimental.pallas import tpu as pltpu
from jax.experimental.pallas import tpu_sc as plsc

NUM_SAMPLES_ = 32

_NC = 2
_NS = 16
_L = 16

_B, _CX, _N = 16, 3, 4096
_C = 256
_S = NUM_SAMPLES_
_G = _N // _S
_GS = _G * _S

_TILES = _NC * _NS
_CPT = _C // (_TILES // _B)
_TR = 8
_NGROUPS = _CPT // _TR
_HG = 4


def _sc_body(ptile_hbm, xyz_hbm, idx2_hbm, idx1_hbm, idxrep_hbm,
             gpts_hbm, nxyz_hbm, gxyz_hbm,
             idx_v, in0, in1, out0, out1, idx1_v,
             nxyz_v, sin0, sin1, sout0, sout1):
    cid = lax.axis_index("c")
    sid = lax.axis_index("s")
    wid = sid * _NC + cid
    b = wid // 2
    tr0 = (wid % 2) * _NGROUPS
    c0 = (wid % 2) * _CPT

    pltpu.sync_copy(idx2_hbm.at[b], idx_v)

    coff = [jnp.full((_L,), cs * 128, dtype=jnp.int32) for cs in range(_TR)]
    ins, outs = [in0, in1], [out0, out1]
    sins, souts = [sin0, sin1], [sout0, sout1]

    def start_in(g, bb):
        pltpu.async_copy(ptile_hbm.at[b, tr0 + g], ins[bb], sins[bb])

    def wait_in(g, bb):
        pltpu.make_async_copy(ptile_hbm.at[b, tr0 + g], ins[bb],
                              sins[bb]).wait()

    def start_out(g, h, ob):
        for r in range(_HG):
            c = c0 + g * _TR + h * _HG + r
            pltpu.async_copy(outs[ob].at[pl.ds(r * _S, _S), :],
                             gpts_hbm.at[b, c], souts[ob])

    def wait_out(g, h, ob):
        for r in range(_HG):
            c = c0 + g * _TR + h * _HG + r
            pltpu.make_async_copy(outs[ob].at[pl.ds(r * _S, _S), :],
                                  gpts_hbm.at[b, c], souts[ob]).wait()

    start_in(0, 0)
    start_in(1, 1)

    def pipe(i, carry):
        for bb in range(2):
            g = 2 * i + bb
            wait_in(g, bb)
            for h in range(2):
                @pl.when(g >= 1)
                def _(g=g, h=h):
                    wait_out(g - 1, h, h)

                def inner(j, c2, bb=bb, h=h):
                    idxv = idx_v[pl.ds(j * _L, _L)]
                    row = j // (_G // _L)
                    g0 = (j % (_G // _L)) * _L
                    for r in range(_HG):
                        outs[h][r * _S + row, pl.ds(g0, _L)] = (
                            plsc.load_gather(
                                ins[bb], [idxv + coff[h * _HG + r]]))
                    return c2

                lax.fori_loop(0, _GS // _L, inner, 0, unroll=4)
                start_out(g, h, h)

            @pl.when(g + 2 < _NGROUPS)
            def _(bb=bb, g=g):
                start_in(g + 2, bb)
        return carry

    lax.fori_loop(0, _NGROUPS // 2, pipe, 0)
    wait_out(_NGROUPS - 1, 0, 0)
    wait_out(_NGROUPS - 1, 1, 1)

    @pl.when(wid < _B)
    def _():
        b2 = wid
        for k in range(_CX):
            pltpu.sync_copy(xyz_hbm.at[b2, k], in0.at[pl.ds(k * _N, _N)])
        pltpu.sync_copy(idx1_hbm.at[b2], idx1_v)
        pltpu.sync_copy(idxrep_hbm.at[b2], idx_v)
        koff = [jnp.full((_L,), k * _N, dtype=jnp.int32) for k in range(_CX)]
        for j in range(_G // _L):
            idxv = idx1_v[pl.ds(j * _L, _L)]
            for k in range(_CX):
                nxyz_v[pl.ds(k * _G + j * _L, _L)] = plsc.load_gather(
                    in0, [idxv + koff[k]])
        for k in range(_CX):
            pltpu.sync_copy(nxyz_v.at[pl.ds(k * _G, _G)], nxyz_hbm.at[b2, k])

        def gx_inner(j, c2):
            idxv = idx_v[pl.ds(j * _L, _L)]
            row = j // (_G // _L)
            g0 = (j % (_G // _L)) * _L
            for k in range(_CX):
                out0[k * _S + row, pl.ds(g0, _L)] = plsc.load_gather(
                    in0, [idxv + koff[k]])
            return c2

        lax.fori_loop(0, _GS // _L, gx_inner, 0, unroll=4)
        for k in range(_CX):
            pltpu.sync_copy(out0.at[pl.ds(k * _S, _S), :],
                            gxyz_hbm.at[b2, k])


@jax.jit
def kernel(xyz, points):
    B, C, N = points.shape
    idx_key = jax.random.key(42)
    k1, k2 = jax.random.split(idx_key)
    idx1 = jax.random.randint(k1, (B, _G), 0, N).astype(jnp.int32)
    idx2 = jax.random.randint(k2, (B, _G, _S), 0, N).astype(jnp.int32)
    idxT = idx2.transpose(0, 2, 1).reshape(B, _GS)
    idx2f = ((idxT >> 7) << 10) + (idxT & 127)
    idxrep = jnp.tile(idx1, (1, _S))

    ptile = points.reshape(B, _C // _TR, _TR, _N // 128, 128)
    ptile = ptile.transpose(0, 1, 3, 2, 4).reshape(B, _C // _TR, _TR * _N)

    mesh = plsc.VectorSubcoreMesh(
        core_axis_name="c", subcore_axis_name="s",
        num_cores=_NC, num_subcores=_NS)
    run = pl.kernel(
        _sc_body,
        out_type=(
            jax.ShapeDtypeStruct((B, C, _S, _G), jnp.float32),
            jax.ShapeDtypeStruct((B, _CX, _G), jnp.float32),
            jax.ShapeDtypeStruct((B, _CX, _S, _G), jnp.float32),
        ),
        mesh=mesh,
        compiler_params=pltpu.CompilerParams(
            needs_layout_passes=False, use_tc_tiling_on_sc=False),
        scratch_types=[
            pltpu.VMEM((_GS,), jnp.int32),
            pltpu.VMEM((_TR * _N,), jnp.float32),
            pltpu.VMEM((_TR * _N,), jnp.float32),
            pltpu.VMEM((_HG * _S, _G), jnp.float32),
            pltpu.VMEM((_HG * _S, _G), jnp.float32),
            pltpu.VMEM((_G,), jnp.int32),
            pltpu.VMEM((_CX * _G,), jnp.float32),
            pltpu.SemaphoreType.DMA,
            pltpu.SemaphoreType.DMA,
            pltpu.SemaphoreType.DMA,
            pltpu.SemaphoreType.DMA,
        ],
    )
    gpts_t, new_xyz, gxyz_t = run(ptile, xyz, idx2f, idx1, idxrep)
    grouped_points = gpts_t.transpose(0, 1, 3, 2)
    grouped_xyz = gxyz_t.transpose(0, 1, 3, 2)
    return (new_xyz, grouped_xyz, grouped_points)


# speedup vs baseline: 6.1978x; 3.0387x over previous
import jax
import jax.numpy as jnp
import numpy as np
from jax import lax
from jax.experimental import pallas as pl
from jax.experimental.pallas import tpu as pltpu
from jax.experimental.pallas import tpu_sc as plsc

NUM_SAMPLES_ = 32

_NC = 2
_NS = 16
_L = 16

_B, _CX, _N = 16, 3, 4096
_C = 256
_S = NUM_SAMPLES_
_G = _N // _S
_GS = _G * _S

_TILES = _NC * _NS
_CPT = _C // (_TILES // _B)
_TR = 8
_NGROUPS = _CPT // _TR
_HG = 4


def _sc_body(ptile_hbm, xyz_hbm, idx2_hbm, idx1_hbm, idxrep_hbm,
             gpts_hbm, nxyz_hbm, gxyz_hbm,
             idx_v, in0, in1, out0, out1, idx1_v,
             nxyz_v, sin0, sin1, sout0, sout1):
    cid = lax.axis_index("c")
    sid = lax.axis_index("s")
    wid = sid * _NC + cid
    b = wid // 2
    tr0 = (wid % 2) * _NGROUPS
    c0 = (wid % 2) * _CPT

    ins, outs = [in0, in1], [out0, out1]
    sins, souts = [sin0, sin1], [sout0, sout1]

    def start_in(g, bb):
        pltpu.async_copy(ptile_hbm.at[b, tr0 + g], ins[bb], sins[bb])

    def wait_in(g, bb):
        pltpu.make_async_copy(ptile_hbm.at[b, tr0 + g], ins[bb],
                              sins[bb]).wait()

    def start_out(g, h, ob):
        for r in range(_HG):
            c = c0 + g * _TR + h * _HG + r
            pltpu.async_copy(outs[ob].at[pl.ds(r * _S, _S), :],
                             gpts_hbm.at[b, c], souts[ob])

    def wait_out(g, h, ob):
        for r in range(_HG):
            c = c0 + g * _TR + h * _HG + r
            pltpu.make_async_copy(outs[ob].at[pl.ds(r * _S, _S), :],
                                  gpts_hbm.at[b, c], souts[ob]).wait()

    start_in(0, 0)
    start_in(1, 1)
    pltpu.sync_copy(idx2_hbm.at[b], idx_v)

    def pipe(i, carry):
        for bb in range(2):
            g = 2 * i + bb
            wait_in(g, bb)
            for h in range(2):
                @pl.when(g >= 1)
                def _(g=g, h=h):
                    wait_out(g - 1, h, h)

                srcs = [
                    ins[bb].at[pl.ds(cs * 128, _TR * _N - cs * 128)]
                    for cs in range(h * _HG, (h + 1) * _HG)
                ]

                @plsc.parallel_loop(0, _GS // _L, unroll=8)
                def _(j, h=h, srcs=srcs):
                    idxv = idx_v[pl.ds(j * _L, _L)]
                    row = j // (_G // _L)
                    g0 = (j % (_G // _L)) * _L
                    for r in range(_HG):
                        outs[h][r * _S + row, pl.ds(g0, _L)] = (
                            plsc.load_gather(srcs[r], [idxv]))

                start_out(g, h, h)

            @pl.when(g + 2 < _NGROUPS)
            def _(bb=bb, g=g):
                start_in(g + 2, bb)
        return carry

    lax.fori_loop(0, _NGROUPS // 2, pipe, 0)
    wait_out(_NGROUPS - 1, 0, 0)
    wait_out(_NGROUPS - 1, 1, 1)

    @pl.when(wid < _B)
    def _():
        b2 = wid
        for k in range(_CX):
            pltpu.sync_copy(xyz_hbm.at[b2, k], in0.at[pl.ds(k * _N, _N)])
        pltpu.sync_copy(idx1_hbm.at[b2], idx1_v)
        pltpu.sync_copy(idxrep_hbm.at[b2], idx_v)
        xsrcs = [in0.at[pl.ds(k * _N, _N)] for k in range(_CX)]
        for j in range(_G // _L):
            idxv = idx1_v[pl.ds(j * _L, _L)]
            for k in range(_CX):
                nxyz_v[pl.ds(k * _G + j * _L, _L)] = plsc.load_gather(
                    xsrcs[k], [idxv])
        for k in range(_CX):
            pltpu.sync_copy(nxyz_v.at[pl.ds(k * _G, _G)], nxyz_hbm.at[b2, k])

        @plsc.parallel_loop(0, _GS // _L, unroll=8)
        def _(j):
            idxv = idx_v[pl.ds(j * _L, _L)]
            row = j // (_G // _L)
            g0 = (j % (_G // _L)) * _L
            for k in range(_CX):
                out0[k * _S + row, pl.ds(g0, _L)] = plsc.load_gather(
                    xsrcs[k], [idxv])
        for k in range(_CX):
            pltpu.sync_copy(out0.at[pl.ds(k * _S, _S), :],
                            gxyz_hbm.at[b2, k])


def _rotl(x, d):
    return ((x << np.uint32(d)) | (x >> np.uint32(32 - d))).astype(np.uint32)


def _tf_block(k0, k1, x0, x1):
    x0 = x0.astype(np.uint32).copy()
    x1 = x1.astype(np.uint32).copy()
    rotations = ((13, 15, 26, 6), (17, 29, 16, 24))
    ks = (np.uint32(k0), np.uint32(k1),
          np.uint32(np.uint32(k0) ^ np.uint32(k1) ^ np.uint32(0x1BD11BDA)))
    x0 += ks[0]
    x1 += ks[1]
    for i in range(5):
        for d in rotations[i % 2]:
            x0 += x1
            x1 = _rotl(x1, d)
            x1 ^= x0
        x0 += ks[(i + 1) % 3]
        x1 += ks[(i + 2) % 3] + np.uint32(i + 1)
    return x0, x1


def _np_split2(keypair):
    b1, b2 = _tf_block(keypair[0], keypair[1],
                       np.zeros(2, np.uint32), np.arange(2, dtype=np.uint32))
    return np.stack([b1, b2], axis=1)


def _np_random_bits32(keypair, shape):
    n = int(np.prod(shape))
    b1, b2 = _tf_block(keypair[0], keypair[1],
                       np.zeros(n, np.uint32), np.arange(n, dtype=np.uint32))
    return (b1 ^ b2).reshape(shape)


def _np_randint(keypair, shape, minval, maxval):
    k1, k2 = _np_split2(keypair)
    higher = _np_random_bits32(k1, shape)
    lower = _np_random_bits32(k2, shape)
    span = np.uint64(maxval - minval)
    multiplier = np.uint64(2 ** 16) % span
    multiplier = (multiplier * multiplier) % span
    offset = ((np.uint64(higher) % span) * multiplier
              + (np.uint64(lower) % span)) % span
    return (np.int64(minval) + offset.astype(np.int64)).astype(np.int32)


def _host_indices():
    base = np.array([0, 42], np.uint32)
    nk1, nk2 = _np_split2(base)
    idx1 = _np_randint(nk1, (_B, _G), 0, _N)
    idx2 = _np_randint(nk2, (_B, _G, _S), 0, _N)
    idxT = idx2.transpose(0, 2, 1).reshape(_B, _GS)
    idx2f = ((idxT >> 7) << 10) + (idxT & 127)
    idxrep = np.tile(idx1, (1, _S))
    return idx1, idx2f.astype(np.int32), idxrep.astype(np.int32)


_IDX1_NP, _IDX2F_NP, _IDXREP_NP = _host_indices()


@jax.jit
def kernel(xyz, points):
    B, C, N = points.shape
    idx1 = jnp.asarray(_IDX1_NP)
    idx2f = jnp.asarray(_IDX2F_NP)
    idxrep = jnp.asarray(_IDXREP_NP)

    ptile = points.reshape(B, _C // _TR, _TR, _N // 128, 128)
    ptile = ptile.transpose(0, 1, 3, 2, 4).reshape(B, _C // _TR, _TR * _N)

    mesh = plsc.VectorSubcoreMesh(
        core_axis_name="c", subcore_axis_name="s",
        num_cores=_NC, num_subcores=_NS)
    run = pl.kernel(
        _sc_body,
        out_type=(
            jax.ShapeDtypeStruct((B, C, _S, _G), jnp.float32),
            jax.ShapeDtypeStruct((B, _CX, _G), jnp.float32),
            jax.ShapeDtypeStruct((B, _CX, _S, _G), jnp.float32),
        ),
        mesh=mesh,
        compiler_params=pltpu.CompilerParams(
            needs_layout_passes=False, use_tc_tiling_on_sc=False),
        scratch_types=[
            pltpu.VMEM((_GS,), jnp.int32),
            pltpu.VMEM((_TR * _N,), jnp.float32),
            pltpu.VMEM((_TR * _N,), jnp.float32),
            pltpu.VMEM((_HG * _S, _G), jnp.float32),
            pltpu.VMEM((_HG * _S, _G), jnp.float32),
            pltpu.VMEM((_G,), jnp.int32),
            pltpu.VMEM((_CX * _G,), jnp.float32),
            pltpu.SemaphoreType.DMA,
            pltpu.SemaphoreType.DMA,
            pltpu.SemaphoreType.DMA,
            pltpu.SemaphoreType.DMA,
        ],
    )
    gpts_t, new_xyz, gxyz_t = run(ptile, xyz, idx2f, idx1, idxrep)
    grouped_points = gpts_t.transpose(0, 1, 3, 2)
    grouped_xyz = gxyz_t.transpose(0, 1, 3, 2)
    return (new_xyz, grouped_xyz, grouped_points)
